# baseline (speedup 1.0000x reference)
import functools

import jax
import jax.numpy as jnp
from jax import lax
from jax.experimental import pallas as pl
from jax.experimental.pallas import tpu as pltpu
from jax.experimental.pallas import tpu_sc as plsc

N = 10000
E = 320000
D = 128
HH = 64
NP = 10240
NTILES = 16
ROWS_PER_TILE = NP // NTILES
CH = 128
NCHUNK = 160
EDGES_PER_TILE = NCHUNK * CH
E_PAD = EDGES_PER_TILE * NTILES
NBUF = 8
AHEAD = 4

_SELU_ALPHA = 1.6732632423543772
_SELU_SCALE = 1.0507009873554805


@functools.partial(
    pl.kernel,
    mesh=plsc.VectorSubcoreMesh(core_axis_name="c", subcore_axis_name="s"),
    out_type=(
        jax.ShapeDtypeStruct((NP, HH), jnp.float32),
        jax.ShapeDtypeStruct((NP, HH), jnp.float32),
    ),
    scratch_types=[
        pltpu.VMEM_SHARED((NP, HH), jnp.float32),
        pltpu.VMEM((NCHUNK, CH), jnp.int32),
        pltpu.VMEM((NBUF, CH), jnp.int32),
        pltpu.VMEM((NBUF, CH, HH), jnp.float32),
        pltpu.SemaphoreType.DMA((NBUF,)),
        pltpu.SemaphoreType.DMA((NBUF,)),
        pltpu.SemaphoreType.DMA((NBUF,)),
    ],
    compiler_params=pltpu.CompilerParams(use_tc_tiling_on_sc=False),
)
def _agg(ha, hb, src, dst, oa, ob, acc, idx_s, idx_d, rows, g_sem, s_sem,
         d_sem):
    c = lax.axis_index("c")
    s = lax.axis_index("s")

    def body(table, out):
        r0 = s * ROWS_PER_TILE
        pltpu.sync_copy(table.at[pl.ds(r0, ROWS_PER_TILE)],
                        acc.at[pl.ds(r0, ROWS_PER_TILE)])
        c0 = s * NCHUNK
        pltpu.sync_copy(src.at[pl.ds(c0, NCHUNK)], idx_s)
        plsc.subcore_barrier()

        for b in range(AHEAD):
            pltpu.async_copy(dst.at[c0 + b], idx_d.at[b], d_sem.at[b])
            pltpu.async_copy(table.at[idx_s.at[b]], rows.at[b], g_sem.at[b])

        def group(g, carry):
            base = g * NBUF
            for b in range(NBUF):
                j = base + b
                pltpu.make_async_copy(table.at[idx_s.at[0]], rows.at[b],
                                      g_sem.at[b]).wait()
                pltpu.make_async_copy(dst.at[0], idx_d.at[b],
                                      d_sem.at[b]).wait()
                pltpu.async_copy(rows.at[b], acc.at[idx_d.at[b]],
                                 s_sem.at[b], add=True)
                jp = j + AHEAD
                bp = (b + AHEAD) % NBUF

                @pl.when(jp >= NBUF)
                def _():
                    pltpu.make_async_copy(rows.at[bp], acc.at[idx_d.at[bp]],
                                          s_sem.at[bp]).wait()

                @pl.when(jp < NCHUNK)
                def _():
                    pltpu.async_copy(dst.at[c0 + jp], idx_d.at[bp],
                                     d_sem.at[bp])
                    pltpu.async_copy(table.at[idx_s.at[jp]], rows.at[bp],
                                     g_sem.at[bp])
            return carry

        lax.fori_loop(0, NCHUNK // NBUF, group, 0)

        for i in range(NBUF - AHEAD):
            b = (NCHUNK - NBUF + AHEAD + i) % NBUF
            pltpu.make_async_copy(rows.at[b], acc.at[idx_d.at[b]],
                                  s_sem.at[b]).wait()
        plsc.subcore_barrier()
        pltpu.sync_copy(acc.at[pl.ds(r0, ROWS_PER_TILE)],
                        out.at[pl.ds(r0, ROWS_PER_TILE)])

    @pl.when(c == 0)
    def _():
        body(ha, oa)

    @pl.when(c == 1)
    def _():
        body(hb, ob)


def _mlp_body(oa_ref, ob_ref, w1a_ref, w1b_ref, b1_ref, pa_ref, pb_ref):
    z = (jnp.dot(oa_ref[...], w1a_ref[...], preferred_element_type=jnp.float32)
         + jnp.dot(ob_ref[...], w1b_ref[...], preferred_element_type=jnp.float32)
         + b1_ref[...])
    act = _SELU_SCALE * jnp.where(z > 0, z, _SELU_ALPHA * (jnp.exp(z) - 1.0))
    pa_ref[...] = act[:, :HH]
    pb_ref[...] = act[:, HH:]


def _mlp(oa, ob, w1a, w1b, b1):
    br = 1024
    grid = (NP // br,)
    return pl.pallas_call(
        _mlp_body,
        grid=grid,
        in_specs=[
            pl.BlockSpec((br, HH), lambda i: (i, 0)),
            pl.BlockSpec((br, HH), lambda i: (i, 0)),
            pl.BlockSpec((HH, D), lambda i: (0, 0)),
            pl.BlockSpec((HH, D), lambda i: (0, 0)),
            pl.BlockSpec((1, D), lambda i: (0, 0)),
        ],
        out_specs=[
            pl.BlockSpec((br, HH), lambda i: (i, 0)),
            pl.BlockSpec((br, HH), lambda i: (i, 0)),
        ],
        out_shape=[
            jax.ShapeDtypeStruct((NP, HH), jnp.float32),
            jax.ShapeDtypeStruct((NP, HH), jnp.float32),
        ],
    )(oa, ob, w1a, w1b, b1)


def _out_body(qa_ref, qb_ref, w2a_ref, w2b_ref, b2_ref, o_ref):
    z = (jnp.dot(qa_ref[...], w2a_ref[...], preferred_element_type=jnp.float32)
         + jnp.dot(qb_ref[...], w2b_ref[...], preferred_element_type=jnp.float32)
         + b2_ref[...])
    m = jnp.max(z, axis=1, keepdims=True)
    lse = jnp.log(jnp.sum(jnp.exp(z - m), axis=1, keepdims=True)) + m
    o_ref[...] = z - lse


def _outk(qa, qb, w2a, w2b, b2):
    br = 1000
    grid = (N // br,)
    return pl.pallas_call(
        _out_body,
        grid=grid,
        in_specs=[
            pl.BlockSpec((br, HH), lambda i: (i, 0)),
            pl.BlockSpec((br, HH), lambda i: (i, 0)),
            pl.BlockSpec((HH, D), lambda i: (0, 0)),
            pl.BlockSpec((HH, D), lambda i: (0, 0)),
            pl.BlockSpec((1, D), lambda i: (0, 0)),
        ],
        out_specs=pl.BlockSpec((br, D), lambda i: (i, 0)),
        out_shape=jax.ShapeDtypeStruct((N, D), jnp.float32),
    )(qa, qb, w2a, w2b, b2)


def kernel(x, edge_index, W1, b1, W2, b2):
    src = edge_index[0]
    dst = edge_index[1]
    pad = E_PAD - E
    src_p = jnp.concatenate([src, jnp.zeros((pad,), jnp.int32)])
    dst_p = jnp.concatenate(
        [dst, N + jnp.arange(pad, dtype=jnp.int32) % (NP - N)])
    src2 = src_p.reshape(E_PAD // CH, CH)
    dst2 = dst_p.reshape(E_PAD // CH, CH)
    xa = jnp.pad(x[:, :HH], ((0, NP - N), (0, 0)))
    xb = jnp.pad(x[:, HH:], ((0, NP - N), (0, 0)))

    h1a, h1b = _agg(xa, xb, src2, dst2)
    h2a, h2b = _agg(h1a, h1b, src2, dst2)
    h3a, h3b = _mlp(h2a, h2b, W1[:HH], W1[HH:], b1.reshape(1, D))
    h4a, h4b = _agg(h3a, h3b, src2, dst2)
    return _outk(h4a, h4b, W2[:HH], W2[HH:], b2.reshape(1, D))


# speedup vs baseline: 2.0897x; 2.0897x over previous
import functools

import jax
import jax.numpy as jnp
from jax import lax
from jax.experimental import pallas as pl
from jax.experimental.pallas import tpu as pltpu
from jax.experimental.pallas import tpu_sc as plsc

N = 10000
E = 320000
D = 128
HH = 64
NP = 10240
NTILES = 16
ROWS_PER_TILE = NP // NTILES
CH = 128
NCHUNK = 160
EDGES_PER_TILE = NCHUNK * CH
E_PAD = EDGES_PER_TILE * NTILES
RB = 4
QB = 8
GA = 2
IA = 6

_SELU_ALPHA = 1.6732632423543772
_SELU_SCALE = 1.0507009873554805


@functools.partial(
    pl.kernel,
    mesh=plsc.VectorSubcoreMesh(core_axis_name="c", subcore_axis_name="s"),
    out_type=(
        jax.ShapeDtypeStruct((NP, HH), jnp.float32),
        jax.ShapeDtypeStruct((NP, HH), jnp.float32),
    ),
    scratch_types=[
        pltpu.VMEM_SHARED((NP, HH), jnp.float32),
        pltpu.VMEM_SHARED((NP, HH), jnp.float32),
        pltpu.VMEM((QB, CH), jnp.int32),
        pltpu.VMEM((QB, CH), jnp.int32),
        pltpu.VMEM((RB, CH, HH), jnp.float32),
        pltpu.SemaphoreType.DMA((RB,)),
        pltpu.SemaphoreType.DMA((RB,)),
        pltpu.SemaphoreType.DMA((QB,)),
        pltpu.SemaphoreType.DMA((QB,)),
    ],
    compiler_params=pltpu.CompilerParams(use_tc_tiling_on_sc=False),
)
def _agg(ha, hb, src, dst, oa, ob, tbl, acc, idx_s, idx_d, rows, g_sem,
         s_sem, i_sem, d_sem):
    c = lax.axis_index("c")
    s = lax.axis_index("s")

    def body(table, out):
        r0 = s * ROWS_PER_TILE
        pltpu.sync_copy(table.at[pl.ds(r0, ROWS_PER_TILE)],
                        tbl.at[pl.ds(r0, ROWS_PER_TILE)])
        pltpu.sync_copy(table.at[pl.ds(r0, ROWS_PER_TILE)],
                        acc.at[pl.ds(r0, ROWS_PER_TILE)])
        c0 = s * NCHUNK
        plsc.subcore_barrier()

        for k in range(IA):
            pltpu.async_copy(src.at[c0 + k], idx_s.at[k], i_sem.at[k])
            pltpu.async_copy(dst.at[c0 + k], idx_d.at[k], d_sem.at[k])
        for k in range(GA):
            pltpu.make_async_copy(src.at[0], idx_s.at[k], i_sem.at[k]).wait()
            pltpu.async_copy(tbl.at[idx_s.at[k]], rows.at[k], g_sem.at[k])

        def group(g, carry):
            base = g * QB
            for u in range(QB):
                j = base + u
                b = u % RB
                pltpu.make_async_copy(tbl.at[idx_s.at[0]], rows.at[b],
                                      g_sem.at[b]).wait()
                pltpu.make_async_copy(dst.at[0], idx_d.at[u],
                                      d_sem.at[u]).wait()
                pltpu.async_copy(rows.at[b], acc.at[idx_d.at[u]],
                                 s_sem.at[b], add=True)

                jp = j + GA
                bp = (u + GA) % RB
                up = (u + GA) % QB

                @pl.when(jp < NCHUNK)
                def _():
                    @pl.when(jp >= RB)
                    def _():
                        pltpu.make_async_copy(rows.at[bp],
                                              acc.at[idx_d.at[up]],
                                              s_sem.at[bp]).wait()
                    pltpu.make_async_copy(src.at[0], idx_s.at[up],
                                          i_sem.at[up]).wait()
                    pltpu.async_copy(tbl.at[idx_s.at[up]], rows.at[bp],
                                     g_sem.at[bp])

                ji = j + IA
                ui = (u + IA) % QB

                @pl.when(ji < NCHUNK)
                def _():
                    pltpu.async_copy(src.at[c0 + ji], idx_s.at[ui],
                                     i_sem.at[ui])
                    pltpu.async_copy(dst.at[c0 + ji], idx_d.at[ui],
                                     d_sem.at[ui])
            return carry

        lax.fori_loop(0, NCHUNK // QB, group, 0)

        for i in range(RB):
            k = NCHUNK - RB + i
            pltpu.make_async_copy(rows.at[k % RB], acc.at[idx_d.at[k % QB]],
                                  s_sem.at[k % RB]).wait()
        plsc.subcore_barrier()
        pltpu.sync_copy(acc.at[pl.ds(r0, ROWS_PER_TILE)],
                        out.at[pl.ds(r0, ROWS_PER_TILE)])

    @pl.when(c == 0)
    def _():
        body(ha, oa)

    @pl.when(c == 1)
    def _():
        body(hb, ob)


def _mlp_body(oa_ref, ob_ref, w1a_ref, w1b_ref, b1_ref, pa_ref, pb_ref):
    z = (jnp.dot(oa_ref[...], w1a_ref[...], preferred_element_type=jnp.float32)
         + jnp.dot(ob_ref[...], w1b_ref[...], preferred_element_type=jnp.float32)
         + b1_ref[...])
    act = _SELU_SCALE * jnp.where(z > 0, z, _SELU_ALPHA * (jnp.exp(z) - 1.0))
    pa_ref[...] = act[:, :HH]
    pb_ref[...] = act[:, HH:]


def _mlp(oa, ob, w1a, w1b, b1):
    br = 1024
    grid = (NP // br,)
    return pl.pallas_call(
        _mlp_body,
        grid=grid,
        in_specs=[
            pl.BlockSpec((br, HH), lambda i: (i, 0)),
            pl.BlockSpec((br, HH), lambda i: (i, 0)),
            pl.BlockSpec((HH, D), lambda i: (0, 0)),
            pl.BlockSpec((HH, D), lambda i: (0, 0)),
            pl.BlockSpec((1, D), lambda i: (0, 0)),
        ],
        out_specs=[
            pl.BlockSpec((br, HH), lambda i: (i, 0)),
            pl.BlockSpec((br, HH), lambda i: (i, 0)),
        ],
        out_shape=[
            jax.ShapeDtypeStruct((NP, HH), jnp.float32),
            jax.ShapeDtypeStruct((NP, HH), jnp.float32),
        ],
    )(oa, ob, w1a, w1b, b1)


def _out_body(qa_ref, qb_ref, w2a_ref, w2b_ref, b2_ref, o_ref):
    z = (jnp.dot(qa_ref[...], w2a_ref[...], preferred_element_type=jnp.float32)
         + jnp.dot(qb_ref[...], w2b_ref[...], preferred_element_type=jnp.float32)
         + b2_ref[...])
    m = jnp.max(z, axis=1, keepdims=True)
    lse = jnp.log(jnp.sum(jnp.exp(z - m), axis=1, keepdims=True)) + m
    o_ref[...] = z - lse


def _outk(qa, qb, w2a, w2b, b2):
    br = 1000
    grid = (N // br,)
    return pl.pallas_call(
        _out_body,
        grid=grid,
        in_specs=[
            pl.BlockSpec((br, HH), lambda i: (i, 0)),
            pl.BlockSpec((br, HH), lambda i: (i, 0)),
            pl.BlockSpec((HH, D), lambda i: (0, 0)),
            pl.BlockSpec((HH, D), lambda i: (0, 0)),
            pl.BlockSpec((1, D), lambda i: (0, 0)),
        ],
        out_specs=pl.BlockSpec((br, D), lambda i: (i, 0)),
        out_shape=jax.ShapeDtypeStruct((N, D), jnp.float32),
    )(qa, qb, w2a, w2b, b2)


def kernel(x, edge_index, W1, b1, W2, b2):
    src = edge_index[0]
    dst = edge_index[1]
    pad = E_PAD - E
    src_p = jnp.concatenate([src, jnp.zeros((pad,), jnp.int32)])
    dst_p = jnp.concatenate(
        [dst, N + jnp.arange(pad, dtype=jnp.int32) % (NP - N)])
    src2 = src_p.reshape(E_PAD // CH, CH)
    dst2 = dst_p.reshape(E_PAD // CH, CH)
    xa = jnp.pad(x[:, :HH], ((0, NP - N), (0, 0)))
    xb = jnp.pad(x[:, HH:], ((0, NP - N), (0, 0)))

    h1a, h1b = _agg(xa, xb, src2, dst2)
    h2a, h2b = _agg(h1a, h1b, src2, dst2)
    h3a, h3b = _mlp(h2a, h2b, W1[:HH], W1[HH:], b1.reshape(1, D))
    h4a, h4b = _agg(h3a, h3b, src2, dst2)
    return _outk(h4a, h4b, W2[:HH], W2[HH:], b2.reshape(1, D))


# speedup vs baseline: 2.4412x; 1.1682x over previous
import functools

import jax
import jax.numpy as jnp
from jax import lax
from jax.experimental import pallas as pl
from jax.experimental.pallas import tpu as pltpu
from jax.experimental.pallas import tpu_sc as plsc

N = 10000
E = 320000
D = 128
HH = 64
NP = 10240
NTILES = 16
ROWS_PER_TILE = NP // NTILES
LAST_ROWS = N - (NTILES - 1) * ROWS_PER_TILE
CH = 128
NCH_FULL = 157
E_PT = NCH_FULL * CH
NCH_LAST = (E - (NTILES - 1) * E_PT) // CH
RB = 4
QB = 8
GA = 2
IA = 6

_SELU_ALPHA = 1.6732632423543772
_SELU_SCALE = 1.0507009873554805

_SC_SCRATCH = [
    pltpu.VMEM_SHARED((NP, HH), jnp.float32),
    pltpu.VMEM_SHARED((NP, HH), jnp.float32),
    pltpu.VMEM((QB, CH), jnp.int32),
    pltpu.VMEM((QB, CH), jnp.int32),
    pltpu.VMEM((RB, CH, HH), jnp.float32),
    pltpu.SemaphoreType.DMA((RB,)),
    pltpu.SemaphoreType.DMA((RB,)),
    pltpu.SemaphoreType.DMA((QB,)),
    pltpu.SemaphoreType.DMA((QB,)),
]


def _edge_pass(ei, tb, ab, e0, nch, idx_s, idx_d, rows, g_sem, s_sem,
               i_sem, d_sem):
    for k in range(IA):
        pltpu.async_copy(ei.at[0, pl.ds(e0 + k * CH, CH)], idx_s.at[k],
                         i_sem.at[k])
        pltpu.async_copy(ei.at[1, pl.ds(e0 + k * CH, CH)], idx_d.at[k],
                         d_sem.at[k])
    for k in range(GA):
        pltpu.make_async_copy(ei.at[0, pl.ds(0, CH)], idx_s.at[k],
                              i_sem.at[k]).wait()
        pltpu.async_copy(tb.at[idx_s.at[k]], rows.at[k], g_sem.at[k])

    def step(j, carry):
        u = lax.rem(j, QB)
        b = lax.rem(j, RB)
        pltpu.make_async_copy(tb.at[idx_s.at[0]], rows.at[b],
                              g_sem.at[b]).wait()
        pltpu.make_async_copy(ei.at[1, pl.ds(0, CH)], idx_d.at[u],
                              d_sem.at[u]).wait()
        pltpu.async_copy(rows.at[b], ab.at[idx_d.at[u]], s_sem.at[b],
                         add=True)

        jp = j + GA

        @pl.when(jp < nch)
        def _():
            up = lax.rem(jp, QB)
            bp = lax.rem(jp, RB)

            @pl.when(jp >= RB)
            def _():
                pltpu.make_async_copy(rows.at[bp], ab.at[idx_d.at[up]],
                                      s_sem.at[bp]).wait()

            pltpu.make_async_copy(ei.at[0, pl.ds(0, CH)], idx_s.at[up],
                                  i_sem.at[up]).wait()
            pltpu.async_copy(tb.at[idx_s.at[up]], rows.at[bp], g_sem.at[bp])

        ji = j + IA

        @pl.when(ji < nch)
        def _():
            ui = lax.rem(ji, QB)
            pltpu.async_copy(ei.at[0, pl.ds(e0 + ji * CH, CH)],
                             idx_s.at[ui], i_sem.at[ui])
            pltpu.async_copy(ei.at[1, pl.ds(e0 + ji * CH, CH)],
                             idx_d.at[ui], d_sem.at[ui])
        return carry

    lax.fori_loop(0, nch, step, 0)

    for i in range(RB):
        k = nch - RB + i
        pltpu.make_async_copy(rows.at[lax.rem(k, RB)],
                              ab.at[idx_d.at[lax.rem(k, QB)]],
                              s_sem.at[lax.rem(k, RB)]).wait()


@functools.partial(
    pl.kernel,
    mesh=plsc.VectorSubcoreMesh(core_axis_name="c", subcore_axis_name="s"),
    out_type=(
        jax.ShapeDtypeStruct((NP, HH), jnp.float32),
        jax.ShapeDtypeStruct((NP, HH), jnp.float32),
    ),
    scratch_types=list(_SC_SCRATCH),
    compiler_params=pltpu.CompilerParams(use_tc_tiling_on_sc=False),
)
def _agg12(x, ei, oa, ob, bufa, bufb, idx_s, idx_d, rows, g_sem, s_sem,
           i_sem, d_sem):
    c = lax.axis_index("c")
    s = lax.axis_index("s")
    r0 = s * ROWS_PER_TILE
    col = c * HH
    last = NTILES - 1

    @pl.when(s < last)
    def _():
        pltpu.sync_copy(x.at[pl.ds(r0, ROWS_PER_TILE), pl.ds(col, HH)],
                        bufa.at[pl.ds(r0, ROWS_PER_TILE)])
        pltpu.sync_copy(x.at[pl.ds(r0, ROWS_PER_TILE), pl.ds(col, HH)],
                        bufb.at[pl.ds(r0, ROWS_PER_TILE)])

    @pl.when(s == last)
    def _():
        pltpu.sync_copy(x.at[pl.ds(r0, LAST_ROWS), pl.ds(col, HH)],
                        bufa.at[pl.ds(r0, LAST_ROWS)])
        pltpu.sync_copy(x.at[pl.ds(r0, LAST_ROWS), pl.ds(col, HH)],
                        bufb.at[pl.ds(r0, LAST_ROWS)])

    nch = jnp.where(s == last, NCH_LAST, NCH_FULL)
    e0 = s * E_PT
    plsc.subcore_barrier()
    _edge_pass(ei, bufa, bufb, e0, nch, idx_s, idx_d, rows, g_sem, s_sem,
               i_sem, d_sem)
    plsc.subcore_barrier()
    for k in range(ROWS_PER_TILE // CH):
        rr = r0 + k * CH
        pltpu.sync_copy(bufb.at[pl.ds(rr, CH)], rows.at[0])
        pltpu.sync_copy(rows.at[0], bufa.at[pl.ds(rr, CH)])
    plsc.subcore_barrier()
    _edge_pass(ei, bufb, bufa, e0, nch, idx_s, idx_d, rows, g_sem, s_sem,
               i_sem, d_sem)
    plsc.subcore_barrier()

    @pl.when(c == 0)
    def _():
        pltpu.sync_copy(bufa.at[pl.ds(r0, ROWS_PER_TILE)],
                        oa.at[pl.ds(r0, ROWS_PER_TILE)])

    @pl.when(c == 1)
    def _():
        pltpu.sync_copy(bufa.at[pl.ds(r0, ROWS_PER_TILE)],
                        ob.at[pl.ds(r0, ROWS_PER_TILE)])


@functools.partial(
    pl.kernel,
    mesh=plsc.VectorSubcoreMesh(core_axis_name="c", subcore_axis_name="s"),
    out_type=(
        jax.ShapeDtypeStruct((NP, HH), jnp.float32),
        jax.ShapeDtypeStruct((NP, HH), jnp.float32),
    ),
    scratch_types=list(_SC_SCRATCH),
    compiler_params=pltpu.CompilerParams(use_tc_tiling_on_sc=False),
)
def _agg3(ha, hb, ei, oa, ob, bufa, bufb, idx_s, idx_d, rows, g_sem, s_sem,
          i_sem, d_sem):
    c = lax.axis_index("c")
    s = lax.axis_index("s")
    r0 = s * ROWS_PER_TILE
    last = NTILES - 1

    def stage(table):
        pltpu.sync_copy(table.at[pl.ds(r0, ROWS_PER_TILE)],
                        bufa.at[pl.ds(r0, ROWS_PER_TILE)])
        pltpu.sync_copy(table.at[pl.ds(r0, ROWS_PER_TILE)],
                        bufb.at[pl.ds(r0, ROWS_PER_TILE)])

    @pl.when(c == 0)
    def _():
        stage(ha)

    @pl.when(c == 1)
    def _():
        stage(hb)

    nch = jnp.where(s == last, NCH_LAST, NCH_FULL)
    e0 = s * E_PT
    plsc.subcore_barrier()
    _edge_pass(ei, bufa, bufb, e0, nch, idx_s, idx_d, rows, g_sem, s_sem,
               i_sem, d_sem)
    plsc.subcore_barrier()

    @pl.when(c == 0)
    def _():
        pltpu.sync_copy(bufb.at[pl.ds(r0, ROWS_PER_TILE)],
                        oa.at[pl.ds(r0, ROWS_PER_TILE)])

    @pl.when(c == 1)
    def _():
        pltpu.sync_copy(bufb.at[pl.ds(r0, ROWS_PER_TILE)],
                        ob.at[pl.ds(r0, ROWS_PER_TILE)])


def _mlp_body(oa_ref, ob_ref, w1_ref, b1_ref, pa_ref, pb_ref):
    z = (jnp.dot(oa_ref[...], w1_ref[:HH, :],
                 preferred_element_type=jnp.float32)
         + jnp.dot(ob_ref[...], w1_ref[HH:, :],
                   preferred_element_type=jnp.float32)
         + b1_ref[...])
    act = _SELU_SCALE * jnp.where(z > 0, z, _SELU_ALPHA * (jnp.exp(z) - 1.0))
    pa_ref[...] = act[:, :HH]
    pb_ref[...] = act[:, HH:]


def _mlp(oa, ob, w1, b1):
    br = 1024
    grid = (NP // br,)
    return pl.pallas_call(
        _mlp_body,
        grid=grid,
        in_specs=[
            pl.BlockSpec((br, HH), lambda i: (i, 0)),
            pl.BlockSpec((br, HH), lambda i: (i, 0)),
            pl.BlockSpec((D, D), lambda i: (0, 0)),
            pl.BlockSpec((1, D), lambda i: (0, 0)),
        ],
        out_specs=[
            pl.BlockSpec((br, HH), lambda i: (i, 0)),
            pl.BlockSpec((br, HH), lambda i: (i, 0)),
        ],
        out_shape=[
            jax.ShapeDtypeStruct((NP, HH), jnp.float32),
            jax.ShapeDtypeStruct((NP, HH), jnp.float32),
        ],
    )(oa, ob, w1, b1)


def _out_body(qa_ref, qb_ref, w2_ref, b2_ref, o_ref):
    z = (jnp.dot(qa_ref[...], w2_ref[:HH, :],
                 preferred_element_type=jnp.float32)
         + jnp.dot(qb_ref[...], w2_ref[HH:, :],
                   preferred_element_type=jnp.float32)
         + b2_ref[...])
    m = jnp.max(z, axis=1, keepdims=True)
    lse = jnp.log(jnp.sum(jnp.exp(z - m), axis=1, keepdims=True)) + m
    o_ref[...] = z - lse


def _outk(qa, qb, w2, b2):
    br = 1000
    grid = (N // br,)
    return pl.pallas_call(
        _out_body,
        grid=grid,
        in_specs=[
            pl.BlockSpec((br, HH), lambda i: (i, 0)),
            pl.BlockSpec((br, HH), lambda i: (i, 0)),
            pl.BlockSpec((D, D), lambda i: (0, 0)),
            pl.BlockSpec((1, D), lambda i: (0, 0)),
        ],
        out_specs=pl.BlockSpec((br, D), lambda i: (i, 0)),
        out_shape=jax.ShapeDtypeStruct((N, D), jnp.float32),
    )(qa, qb, w2, b2)


def kernel(x, edge_index, W1, b1, W2, b2):
    h2a, h2b = _agg12(x, edge_index)
    h3a, h3b = _mlp(h2a, h2b, W1, b1.reshape(1, D))
    h4a, h4b = _agg3(h3a, h3b, edge_index)
    return _outk(h4a, h4b, W2, b2.reshape(1, D))
